# baseline (speedup 1.0000x reference)
import jax
import jax.numpy as jnp
from jax import lax
from jax.experimental import pallas as pl
from jax.experimental.pallas import tpu as pltpu

_N_BINS = 10
_ROWS = 500000
_COLS = 128
_CHUNK = 10000


def _ece_body(lo_ref, hi_ref, x_ref, lab_ref, ece_ref, acc_ref):
    pid = pl.program_id(0)
    nsteps = pl.num_programs(0)

    @pl.when(pid == 0)
    def _init():
        acc_ref[...] = jnp.zeros_like(acc_ref)

    x = x_ref[...]
    lab = lab_ref[...]
    m = jnp.max(x, axis=1, keepdims=True)
    conf = jnp.exp(m)
    col = lax.broadcasted_iota(jnp.int32, x.shape, 1)
    am = jnp.min(jnp.where(x == m, col, _COLS), axis=1, keepdims=True)
    acc = (am == lab).astype(jnp.float32)

    lo = lo_ref[...]
    hi = hi_ref[...]
    mask = ((conf > lo) & (conf <= hi)).astype(jnp.float32)
    acc_ref[0:1, :] += jnp.sum(mask, axis=0, keepdims=True)
    acc_ref[1:2, :] += jnp.sum(mask * conf, axis=0, keepdims=True)
    acc_ref[2:3, :] += jnp.sum(mask * acc, axis=0, keepdims=True)

    @pl.when(pid == nsteps - 1)
    def _fin():
        cnt = acc_ref[0:1, :]
        sconf = acc_ref[1:2, :]
        sacc = acc_ref[2:3, :]
        safe = jnp.maximum(cnt, 1.0)
        prop = cnt * (1.0 / _ROWS)
        contrib = jnp.abs(sconf / safe - sacc / safe) * prop
        contrib = jnp.where(prop > 0.0, contrib, 0.0)
        ece_ref[...] = jnp.sum(contrib, axis=1, keepdims=True)


def _bin_bounds():
    b = jnp.linspace(0.0, 1.0, _N_BINS + 1)
    lane = jnp.arange(_COLS)
    lo = jnp.where(lane < _N_BINS, b[jnp.minimum(lane, _N_BINS - 1)], 2.0)
    hi = jnp.where(lane < _N_BINS, b[jnp.minimum(lane + 1, _N_BINS)], 2.0)
    return lo.reshape(1, _COLS).astype(jnp.float32), hi.reshape(1, _COLS).astype(jnp.float32)


def kernel(logits, labels):
    lo, hi = _bin_bounds()
    lab2d = labels.astype(jnp.int32).reshape(_ROWS, 1)
    grid = _ROWS // _CHUNK
    ece = pl.pallas_call(
        _ece_body,
        grid=(grid,),
        in_specs=[
            pl.BlockSpec((1, _COLS), lambda i: (0, 0)),
            pl.BlockSpec((1, _COLS), lambda i: (0, 0)),
            pl.BlockSpec((_CHUNK, _COLS), lambda i: (i, 0)),
            pl.BlockSpec((_CHUNK, 1), lambda i: (i, 0)),
        ],
        out_specs=pl.BlockSpec((1, 1), lambda i: (0, 0)),
        out_shape=jax.ShapeDtypeStruct((1, 1), jnp.float32),
        scratch_shapes=[pltpu.VMEM((8, _COLS), jnp.float32)],
    )(lo, hi, logits, lab2d)
    return ece.reshape(1)


# speedup vs baseline: 1.3992x; 1.3992x over previous
import jax
import jax.numpy as jnp
from jax import lax
from jax.experimental import pallas as pl
from jax.experimental.pallas import tpu as pltpu

_N_BINS = 10
_ROWS = 500000
_COLS = 128
_CHUNK = 10000


def _ece_body(lo_ref, hi_ref, x_ref, lab_ref, ece_ref, acc_ref):
    pid = pl.program_id(0)
    nsteps = pl.num_programs(0)

    @pl.when(pid == 0)
    def _init():
        acc_ref[...] = jnp.zeros_like(acc_ref)

    x = x_ref[...]
    m = jnp.max(x, axis=1, keepdims=True)

    @pl.when(jnp.any(m <= 0.0))
    def _bin_chunk():
        lab = lab_ref[...]
        conf = jnp.exp(m)
        col = lax.broadcasted_iota(jnp.int32, x.shape, 1)
        am = jnp.min(jnp.where(x == m, col, _COLS), axis=1, keepdims=True)
        acc = (am == lab).astype(jnp.float32)
        lo = lo_ref[...]
        hi = hi_ref[...]
        mask = ((conf > lo) & (conf <= hi)).astype(jnp.float32)
        acc_ref[0:1, :] += jnp.sum(mask, axis=0, keepdims=True)
        acc_ref[1:2, :] += jnp.sum(mask * conf, axis=0, keepdims=True)
        acc_ref[2:3, :] += jnp.sum(mask * acc, axis=0, keepdims=True)

    @pl.when(pid == nsteps - 1)
    def _fin():
        cnt = acc_ref[0:1, :]
        sconf = acc_ref[1:2, :]
        sacc = acc_ref[2:3, :]
        safe = jnp.maximum(cnt, 1.0)
        prop = cnt * (1.0 / _ROWS)
        contrib = jnp.abs(sconf / safe - sacc / safe) * prop
        contrib = jnp.where(prop > 0.0, contrib, 0.0)
        ece_ref[...] = jnp.sum(contrib, axis=1, keepdims=True)


def _bin_bounds():
    b = jnp.linspace(0.0, 1.0, _N_BINS + 1)
    lane = jnp.arange(_COLS)
    lo = jnp.where(lane < _N_BINS, b[jnp.minimum(lane, _N_BINS - 1)], 2.0)
    hi = jnp.where(lane < _N_BINS, b[jnp.minimum(lane + 1, _N_BINS)], 2.0)
    return lo.reshape(1, _COLS).astype(jnp.float32), hi.reshape(1, _COLS).astype(jnp.float32)


def kernel(logits, labels):
    lo, hi = _bin_bounds()
    lab2d = labels.astype(jnp.int32).reshape(_ROWS, 1)
    grid = _ROWS // _CHUNK
    ece = pl.pallas_call(
        _ece_body,
        grid=(grid,),
        in_specs=[
            pl.BlockSpec((1, _COLS), lambda i: (0, 0)),
            pl.BlockSpec((1, _COLS), lambda i: (0, 0)),
            pl.BlockSpec((_CHUNK, _COLS), lambda i: (i, 0)),
            pl.BlockSpec((_CHUNK, 1), lambda i: (i, 0)),
        ],
        out_specs=pl.BlockSpec((1, 1), lambda i: (0, 0)),
        out_shape=jax.ShapeDtypeStruct((1, 1), jnp.float32),
        scratch_shapes=[pltpu.VMEM((8, _COLS), jnp.float32)],
    )(lo, hi, logits, lab2d)
    return ece.reshape(1)
